# baseline (speedup 1.0000x reference)
import functools

import jax
import jax.numpy as jnp
from jax import lax
from jax.experimental import pallas as pl
from jax.experimental.pallas import tpu as pltpu
from jax.experimental.pallas import tpu_sc as plsc

N = 16384
D = 2048
C = 10000
CPAD = 10240
NC = 2
NS = 16
NW = NC * NS
RPW = N // NW
G = 8
NCHUNK = RPW // G
LPT = N // NS
CPT = CPAD // NS


def _fill(ref, n, val, dtype):
  def body(i, _):
    ref[pl.ds(i * 16, 16)] = jnp.full((16,), val, dtype)
    return 0
  lax.fori_loop(0, n // 16, body, 0)


def _sc_center_loss(feat_hbm, label_hbm, centers_hbm, out_hbm,
                    lbl_v, hist_lbl_v, ones_v, cnt_tab_v, recip_v,
                    fbuf0, fbuf1, cbuf0, cbuf1, acc_v, cnt_shared,
                    sem_f0, sem_f1, sem_c0, sem_c1):
  sid = lax.axis_index("s")
  cid = lax.axis_index("c")
  wid = sid * NC + cid
  base = wid * RPW

  def start(t, fb, cb, sf, sc_):
    pltpu.async_copy(feat_hbm.at[pl.ds(base + t * G, G)], fb, sf)
    pltpu.async_copy(centers_hbm.at[lbl_v.at[pl.ds(t * G, G)]], cb, sc_)

  def wait(t, fb, cb, sf, sc_):
    pltpu.make_async_copy(feat_hbm.at[pl.ds(base + t * G, G)], fb, sf).wait()
    pltpu.make_async_copy(
        centers_hbm.at[lbl_v.at[pl.ds(t * G, G)]], cb, sc_).wait()

  pltpu.sync_copy(label_hbm.at[pl.ds(base, RPW)], lbl_v)
  start(0, fbuf0, cbuf0, sem_f0, sem_c0)
  start(1, fbuf1, cbuf1, sem_f1, sem_c1)

  _fill(ones_v, CPT, 0.0, jnp.float32)
  pltpu.sync_copy(ones_v.at[pl.ds(0, CPT)], cnt_shared.at[pl.ds(sid * CPT, CPT)])
  plsc.subcore_barrier()

  pltpu.sync_copy(label_hbm.at[pl.ds(sid * LPT, LPT)], hist_lbl_v)
  _fill(ones_v, LPT, 1.0, jnp.float32)
  pltpu.sync_copy(ones_v, cnt_shared.at[hist_lbl_v], add=True)
  plsc.subcore_barrier()

  pltpu.sync_copy(cnt_shared, cnt_tab_v)

  def recip_body(i, _):
    lc = lbl_v[pl.ds(i * 16, 16)]
    cv = plsc.load_gather(cnt_tab_v, [lc])
    recip_v[pl.ds(i * 16, 16)] = 1.0 / cv
    return 0
  lax.fori_loop(0, RPW // 16, recip_body, 0)

  acc_v[...] = jnp.zeros((16,), jnp.float32)

  def compute(t, fb, cb):
    for r in range(G):
      def col_body(k, accs):
        a0, a1, a2, a3 = accs
        b = k
        d0 = fb[r, pl.ds(b, 16)] - cb[r, pl.ds(b, 16)]
        d1 = fb[r, pl.ds(b + 16, 16)] - cb[r, pl.ds(b + 16, 16)]
        d2 = fb[r, pl.ds(b + 32, 16)] - cb[r, pl.ds(b + 32, 16)]
        d3 = fb[r, pl.ds(b + 48, 16)] - cb[r, pl.ds(b + 48, 16)]
        return (a0 + d0 * d0, a1 + d1 * d1, a2 + d2 * d2, a3 + d3 * d3)

      z = jnp.zeros((16,), jnp.float32)
      a0, a1, a2, a3 = plsc.parallel_loop(
          0, D, 64, unroll=4, carry=(z, z, z, z))(col_body)
      row_acc = (a0 + a1) + (a2 + a3)
      br = plsc.load_gather(
          recip_v, [jnp.full((16,), t * G + r, jnp.int32)])
      acc_v[...] = acc_v[...] + row_acc * br

  def ring_body(i, _):
    t0 = 2 * i
    wait(t0, fbuf0, cbuf0, sem_f0, sem_c0)
    compute(t0, fbuf0, cbuf0)

    @pl.when(t0 + 2 < NCHUNK)
    def _():
      start(t0 + 2, fbuf0, cbuf0, sem_f0, sem_c0)

    t1 = t0 + 1
    wait(t1, fbuf1, cbuf1, sem_f1, sem_c1)
    compute(t1, fbuf1, cbuf1)

    @pl.when(t1 + 2 < NCHUNK)
    def _():
      start(t1 + 2, fbuf1, cbuf1, sem_f1, sem_c1)
    return 0

  lax.fori_loop(0, NCHUNK // 2, ring_body, 0)

  pltpu.sync_copy(acc_v, out_hbm.at[wid])


@functools.partial(jax.jit, static_argnames=())
def _run(feat, label, centers):
  mesh = plsc.VectorSubcoreMesh(core_axis_name="c", subcore_axis_name="s")
  f = pl.kernel(
      _sc_center_loss,
      out_type=jax.ShapeDtypeStruct((NW, 16), jnp.float32),
      mesh=mesh,
      compiler_params=pltpu.CompilerParams(needs_layout_passes=False),
      scratch_types=[
          pltpu.VMEM((RPW,), jnp.int32),
          pltpu.VMEM((LPT,), jnp.int32),
          pltpu.VMEM((LPT,), jnp.float32),
          pltpu.VMEM((CPAD,), jnp.float32),
          pltpu.VMEM((RPW,), jnp.float32),
          pltpu.VMEM((G, D), jnp.float32),
          pltpu.VMEM((G, D), jnp.float32),
          pltpu.VMEM((G, D), jnp.float32),
          pltpu.VMEM((G, D), jnp.float32),
          pltpu.VMEM((16,), jnp.float32),
          pltpu.VMEM_SHARED((CPAD,), jnp.float32),
          pltpu.SemaphoreType.DMA,
          pltpu.SemaphoreType.DMA,
          pltpu.SemaphoreType.DMA,
          pltpu.SemaphoreType.DMA,
      ],
  )
  partials = f(feat, label.astype(jnp.int32), centers)
  return jnp.sum(partials) / jnp.float32(N)


def kernel(feat, label, centers):
  return _run(feat, label, centers)


# speedup vs baseline: 1.1545x; 1.1545x over previous
import functools

import jax
import jax.numpy as jnp
from jax import lax
from jax.experimental import pallas as pl
from jax.experimental.pallas import tpu as pltpu
from jax.experimental.pallas import tpu_sc as plsc

N = 16384
D = 2048
C = 10000
CPAD = 10240
NC = 2
NS = 16
NW = NC * NS
RPW = N // NW
G = 8
NCHUNK = RPW // G
LPT = N // NS
CPT = CPAD // NS


def _fill(ref, n, val, dtype):
  def body(i, _):
    ref[pl.ds(i * 16, 16)] = jnp.full((16,), val, dtype)
    return 0
  lax.fori_loop(0, n // 16, body, 0)


DEPTH = 3


def _sc_center_loss(feat_hbm, label_hbm, centers_hbm, out_hbm,
                    lbl_v, hist_lbl_v, ones_v, cnt_tab_v, recip_v,
                    fbuf0, fbuf1, fbuf2, cbuf0, cbuf1, cbuf2,
                    acc_v, cnt_shared,
                    sem_f0, sem_f1, sem_f2, sem_c0, sem_c1, sem_c2):
  sid = lax.axis_index("s")
  cid = lax.axis_index("c")
  wid = sid * NC + cid
  base = wid * RPW

  bufs = ((fbuf0, cbuf0, sem_f0, sem_c0),
          (fbuf1, cbuf1, sem_f1, sem_c1),
          (fbuf2, cbuf2, sem_f2, sem_c2))

  def start(t, p):
    fb, cb, sf, sc_ = bufs[p]
    pltpu.async_copy(feat_hbm.at[pl.ds(base + t * G, G)], fb, sf)
    pltpu.async_copy(centers_hbm.at[lbl_v.at[pl.ds(t * G, G)]], cb, sc_)

  def wait(t, p):
    fb, cb, sf, sc_ = bufs[p]
    pltpu.make_async_copy(feat_hbm.at[pl.ds(base + t * G, G)], fb, sf).wait()
    pltpu.make_async_copy(
        centers_hbm.at[lbl_v.at[pl.ds(t * G, G)]], cb, sc_).wait()

  pltpu.sync_copy(label_hbm.at[pl.ds(base, RPW)], lbl_v)
  for p in range(DEPTH):
    start(p, p)

  _fill(ones_v, CPT, 0.0, jnp.float32)
  pltpu.sync_copy(ones_v.at[pl.ds(0, CPT)], cnt_shared.at[pl.ds(sid * CPT, CPT)])
  plsc.subcore_barrier()

  pltpu.sync_copy(label_hbm.at[pl.ds(sid * LPT, LPT)], hist_lbl_v)
  _fill(ones_v, LPT, 1.0, jnp.float32)
  pltpu.sync_copy(ones_v, cnt_shared.at[hist_lbl_v], add=True)
  plsc.subcore_barrier()

  pltpu.sync_copy(cnt_shared, cnt_tab_v)

  def recip_body(i, _):
    lc = lbl_v[pl.ds(i * 16, 16)]
    cv = plsc.load_gather(cnt_tab_v, [lc])
    recip_v[pl.ds(i * 16, 16)] = 1.0 / cv
    return 0
  lax.fori_loop(0, RPW // 16, recip_body, 0)

  acc_v[...] = jnp.zeros((16,), jnp.float32)

  def compute(t, p):
    fb, cb = bufs[p][0], bufs[p][1]
    for r in range(G):
      def col_body(k, accs):
        a0, a1, a2, a3 = accs
        b = k
        d0 = fb[r, pl.ds(b, 16)] - cb[r, pl.ds(b, 16)]
        d1 = fb[r, pl.ds(b + 16, 16)] - cb[r, pl.ds(b + 16, 16)]
        d2 = fb[r, pl.ds(b + 32, 16)] - cb[r, pl.ds(b + 32, 16)]
        d3 = fb[r, pl.ds(b + 48, 16)] - cb[r, pl.ds(b + 48, 16)]
        return (a0 + d0 * d0, a1 + d1 * d1, a2 + d2 * d2, a3 + d3 * d3)

      z = jnp.zeros((16,), jnp.float32)
      a0, a1, a2, a3 = lax.fori_loop(0, D // 64,
                                     lambda k, a: col_body(k * 64, a),
                                     (z, z, z, z))
      row_acc = (a0 + a1) + (a2 + a3)
      br = plsc.load_gather(
          recip_v, [jnp.full((16,), t * G + r, jnp.int32)])
      acc_v[...] = acc_v[...] + row_acc * br

  def ring_body(i, _):
    for p in range(DEPTH):
      t = DEPTH * i + p
      wait(t, p)
      compute(t, p)

      @pl.when(t + DEPTH < NCHUNK)
      def _():
        start(t + DEPTH, p)
    return 0

  full = NCHUNK // DEPTH
  lax.fori_loop(0, full, ring_body, 0)
  for t in range(full * DEPTH, NCHUNK):
    wait(t, t % DEPTH)
    compute(t, t % DEPTH)

  pltpu.sync_copy(acc_v, out_hbm.at[wid])


@functools.partial(jax.jit, static_argnames=())
def _run(feat, label, centers):
  mesh = plsc.VectorSubcoreMesh(core_axis_name="c", subcore_axis_name="s")
  f = pl.kernel(
      _sc_center_loss,
      out_type=jax.ShapeDtypeStruct((NW, 16), jnp.float32),
      mesh=mesh,
      compiler_params=pltpu.CompilerParams(needs_layout_passes=False),
      scratch_types=[
          pltpu.VMEM((RPW,), jnp.int32),
          pltpu.VMEM((LPT,), jnp.int32),
          pltpu.VMEM((LPT,), jnp.float32),
          pltpu.VMEM((CPAD,), jnp.float32),
          pltpu.VMEM((RPW,), jnp.float32),
          pltpu.VMEM((G, D), jnp.float32),
          pltpu.VMEM((G, D), jnp.float32),
          pltpu.VMEM((G, D), jnp.float32),
          pltpu.VMEM((G, D), jnp.float32),
          pltpu.VMEM((G, D), jnp.float32),
          pltpu.VMEM((G, D), jnp.float32),
          pltpu.VMEM((16,), jnp.float32),
          pltpu.VMEM_SHARED((CPAD,), jnp.float32),
          pltpu.SemaphoreType.DMA,
          pltpu.SemaphoreType.DMA,
          pltpu.SemaphoreType.DMA,
          pltpu.SemaphoreType.DMA,
          pltpu.SemaphoreType.DMA,
          pltpu.SemaphoreType.DMA,
      ],
  )
  partials = f(feat, label.astype(jnp.int32), centers)
  return jnp.sum(partials) / jnp.float32(N)


def kernel(feat, label, centers):
  return _run(feat, label, centers)


# speedup vs baseline: 1.1912x; 1.0318x over previous
import functools

import jax
import jax.numpy as jnp
from jax import lax
from jax.experimental import pallas as pl
from jax.experimental.pallas import tpu as pltpu
from jax.experimental.pallas import tpu_sc as plsc

N = 16384
D = 2048
C = 10000
CPAD = 10240
NC = 2
NS = 16
NW = NC * NS
RPW = N // NW
G = 8
DEPTH = 3
NCHUNK = RPW // G
LPT = N // NS
CPT = CPAD // NS


def _fill(ref, n, val, dtype):
  def body(i, _):
    ref[pl.ds(i * 16, 16)] = jnp.full((16,), val, dtype)
    return 0
  lax.fori_loop(0, n // 16, body, 0)


def _sc_center_loss(feat_hbm, label_hbm, centers_hbm, out_hbm,
                    lbl_v, hist_lbl_v, ones_v, cnt_tab_v, recip_v,
                    acc_v, cnt_shared, *ring):
  sid = lax.axis_index("s")
  cid = lax.axis_index("c")
  wid = sid * NC + cid
  base = wid * RPW

  fbufs = ring[0:DEPTH]
  cbufs = ring[DEPTH:2 * DEPTH]
  sem_f = ring[2 * DEPTH:3 * DEPTH]
  sem_c = ring[3 * DEPTH:4 * DEPTH]

  def start(t, p):
    pltpu.async_copy(feat_hbm.at[pl.ds(base + t * G, G)], fbufs[p], sem_f[p])
    pltpu.async_copy(
        centers_hbm.at[lbl_v.at[pl.ds(t * G, G)]], cbufs[p], sem_c[p])

  def wait(t, p):
    pltpu.make_async_copy(
        feat_hbm.at[pl.ds(base + t * G, G)], fbufs[p], sem_f[p]).wait()
    pltpu.make_async_copy(
        centers_hbm.at[lbl_v.at[pl.ds(t * G, G)]], cbufs[p], sem_c[p]).wait()

  pltpu.sync_copy(label_hbm.at[pl.ds(base, RPW)], lbl_v)
  for p in range(DEPTH):
    start(p, p)

  _fill(ones_v, CPT, 0.0, jnp.float32)
  pltpu.sync_copy(ones_v.at[pl.ds(0, CPT)],
                  cnt_shared.at[pl.ds(sid * CPT, CPT)])
  plsc.subcore_barrier()

  pltpu.sync_copy(label_hbm.at[pl.ds(sid * LPT, LPT)], hist_lbl_v)
  _fill(ones_v, LPT, 1.0, jnp.float32)
  pltpu.sync_copy(ones_v, cnt_shared.at[hist_lbl_v], add=True)
  plsc.subcore_barrier()

  pltpu.sync_copy(cnt_shared, cnt_tab_v)

  def recip_body(i, _):
    lc = lbl_v[pl.ds(i * 16, 16)]
    cv = plsc.load_gather(cnt_tab_v, [lc])
    recip_v[pl.ds(i * 16, 16)] = 1.0 / cv
    return 0
  lax.fori_loop(0, RPW // 16, recip_body, 0)

  acc_v[...] = jnp.zeros((16,), jnp.float32)

  def compute(t, p):
    fb, cb = fbufs[p], cbufs[p]
    for r in range(G):
      def col_body(k, accs):
        a0, a1, a2, a3 = accs
        b = k * 128
        for u in range(0, 128, 64):
          d0 = fb[r, pl.ds(b + u, 16)] - cb[r, pl.ds(b + u, 16)]
          d1 = fb[r, pl.ds(b + u + 16, 16)] - cb[r, pl.ds(b + u + 16, 16)]
          d2 = fb[r, pl.ds(b + u + 32, 16)] - cb[r, pl.ds(b + u + 32, 16)]
          d3 = fb[r, pl.ds(b + u + 48, 16)] - cb[r, pl.ds(b + u + 48, 16)]
          a0, a1, a2, a3 = (a0 + d0 * d0, a1 + d1 * d1,
                            a2 + d2 * d2, a3 + d3 * d3)
        return (a0, a1, a2, a3)

      z = jnp.zeros((16,), jnp.float32)
      a0, a1, a2, a3 = lax.fori_loop(0, D // 128, col_body, (z, z, z, z))
      row_acc = (a0 + a1) + (a2 + a3)
      br = plsc.load_gather(
          recip_v, [jnp.full((16,), t * G + r, jnp.int32)])
      acc_v[...] = acc_v[...] + row_acc * br

  def ring_body(i, _):
    for p in range(DEPTH):
      t = DEPTH * i + p
      wait(t, p)
      compute(t, p)

      @pl.when(t + DEPTH < NCHUNK)
      def _():
        start(t + DEPTH, p)
    return 0

  full = NCHUNK // DEPTH
  lax.fori_loop(0, full, ring_body, 0)
  for t in range(full * DEPTH, NCHUNK):
    wait(t, t % DEPTH)
    compute(t, t % DEPTH)

  pltpu.sync_copy(acc_v, out_hbm.at[wid])


@functools.partial(jax.jit, static_argnames=())
def _run(feat, label, centers):
  mesh = plsc.VectorSubcoreMesh(core_axis_name="c", subcore_axis_name="s")
  f = pl.kernel(
      _sc_center_loss,
      out_type=jax.ShapeDtypeStruct((NW, 16), jnp.float32),
      mesh=mesh,
      compiler_params=pltpu.CompilerParams(needs_layout_passes=False),
      scratch_types=[
          pltpu.VMEM((RPW,), jnp.int32),
          pltpu.VMEM((LPT,), jnp.int32),
          pltpu.VMEM((LPT,), jnp.float32),
          pltpu.VMEM((CPAD,), jnp.float32),
          pltpu.VMEM((RPW,), jnp.float32),
          pltpu.VMEM((16,), jnp.float32),
          pltpu.VMEM_SHARED((CPAD,), jnp.float32),
      ]
      + [pltpu.VMEM((G, D), jnp.float32)] * (2 * DEPTH)
      + [pltpu.SemaphoreType.DMA] * (2 * DEPTH),
  )
  partials = f(feat, label.astype(jnp.int32), centers)
  return jnp.sum(partials) / jnp.float32(N)


def kernel(feat, label, centers):
  return _run(feat, label, centers)
